# Initial kernel scaffold; baseline (speedup 1.0000x reference)
import jax
import jax.numpy as jnp
from jax.experimental import pallas as pl


def kernel(support, support_label, sample, w1, b1, g1, be1, w2, b2, g2, be2, w3, b3, g3, be3, w4, b4, g4, be4):
    raise NotImplementedError("write your pallas kernel here")


# speedup vs baseline: 1.0091x; 1.0091x over previous
import functools

import jax
import jax.numpy as jnp
from jax.experimental import pallas as pl
from jax.experimental.pallas import tpu as pltpu

WAY, SHOT, QUIRY = 5, 5, 15
B = 8
IMG = 84
EPS = 1e-5
F32 = jnp.float32

N_SUP = B * WAY * SHOT
N_SMP = B * QUIRY

L1G = IMG + 2
L1FLAT = L1G * L1G
L1LANES = 7424
L1TOT = 7680
L1OFF = 87


def _l1_kernel(x_ref, w_ref, b_ref, act_ref, st_ref, xcolT, s86):
    for t in range(9):
        dh, dw = t // 3, t % 3
        s = L1OFF + (dh - 1) * L1G + (dw - 1)
        xcolT[8 * t:8 * t + 8, :] = x_ref[0, :, pl.ds(s, L1LANES)]
    y = jax.lax.dot_general(w_ref[...], xcolT[...],
                            (((0,), (0,)), ((), ())),
                            preferred_element_type=F32)
    y = jnp.maximum(y + b_ref[0][:, None], 0.0)
    yt = jnp.transpose(y, (1, 0))
    s86[...] = yt[0:L1FLAT, :].reshape(L1G, L1G, 64)
    interior = s86[1:1 + IMG, 1:1 + IMG, :]
    act_ref[0] = interior
    st_ref[0] = jnp.stack([jnp.sum(interior, axis=(0, 1)),
                           jnp.sum(interior * interior, axis=(0, 1))])


def _layer_kernel(act_ref, st_in_ref, w_ref, b_ref, g_ref, be_ref,
                  act_out_ref, st_out_ref, xpad, xcol,
                  *, n_img, h_in, h_out):
    cnt = float(n_img * h_in * h_in)
    s = jnp.sum(st_in_ref[...], axis=0)
    mean = s[0] / cnt
    var = s[1] / cnt - mean * mean
    scale = g_ref[0] * jax.lax.rsqrt(var + EPS)
    shift = be_ref[0] - mean * scale

    h2 = h_out

    def pslice(i, j):
        return act_ref[pl.ds(0, 1), pl.ds(i, h2, 2), pl.ds(j, h2, 2), :][0]

    a00, a01, a10, a11 = pslice(0, 0), pslice(0, 1), pslice(1, 0), pslice(1, 1)
    pmax = jnp.maximum(jnp.maximum(a00, a01), jnp.maximum(a10, a11))
    pmin = jnp.minimum(jnp.minimum(a00, a01), jnp.minimum(a10, a11))
    pooled = jnp.where(scale >= 0.0, pmax, pmin)
    x = pooled * scale + shift

    hp = h2 + 2
    zrow = jnp.zeros((hp, 64), F32)
    xpad[0, :, :] = zrow
    xpad[hp - 1, :, :] = zrow
    xpad[:, 0, :] = zrow
    xpad[:, hp - 1, :] = zrow
    xpad[1:1 + h2, 1:1 + h2, :] = x

    m = h2 * h2
    for t in range(9):
        dh, dw = t // 3, t % 3
        xcol[:, 64 * t:64 * (t + 1)] = (
            xpad[dh:dh + h2, dw:dw + h2, :].reshape(m, 64))
    z = jnp.dot(xcol[...], w_ref[...], preferred_element_type=F32)
    z = jnp.maximum(z + b_ref[0], 0.0)
    act_out_ref[0] = z.reshape(h2, h2, 64)
    st_out_ref[0] = jnp.stack([jnp.sum(z, axis=0), jnp.sum(z * z, axis=0)])


def _head_kernel(act_ref, st_in_ref, g_ref, be_ref, f_ref, *, n_img):
    cnt = float(n_img * 10 * 10)
    s = jnp.sum(st_in_ref[...], axis=0)
    mean = s[0] / cnt
    var = s[1] / cnt - mean * mean
    scale = g_ref[0] * jax.lax.rsqrt(var + EPS)
    shift = be_ref[0] - mean * scale
    def pslice(i, j):
        return act_ref[pl.ds(0, 1), pl.ds(i, 5, 2), pl.ds(j, 5, 2), :][0]

    a00, a01, a10, a11 = pslice(0, 0), pslice(0, 1), pslice(1, 0), pslice(1, 1)
    pmax = jnp.maximum(jnp.maximum(a00, a01), jnp.maximum(a10, a11))
    pmin = jnp.minimum(jnp.minimum(a00, a01), jnp.minimum(a10, a11))
    pooled = jnp.where(scale >= 0.0, pmax, pmin)
    x = pooled * scale + shift
    f_ref[0] = x.reshape(25, 64)


def _simplex_kernel(sup_ref, smp_ref, out_ref):
    Sr = sup_ref[0]
    Q = smp_ref[0]
    hi = jax.lax.Precision.HIGHEST
    U = [Sr[5 * i:5 * (i + 1), :] for i in range(5)]
    P = jax.lax.dot_general(Q, Sr, (((1,), (1,)), ((), ())),
                            precision=hi, preferred_element_type=F32)
    Pi = [P[:, 5 * i:5 * (i + 1)] for i in range(5)]
    mm = jnp.sum(Q * Q, axis=1, keepdims=True)
    d = {}
    for i in range(5):
        for j in range(i, 5):
            d[(i, j)] = jnp.sum(U[i] * U[j], axis=1)
            d[(j, i)] = d[(i, j)]

    G = {}
    for i in range(5):
        for j in range(i, 5):
            G[(i, j)] = d[(i, j)][None, :] - Pi[i] - Pi[j] + mm

    def gget(g, i, j):
        return g[(i, j)] if i <= j else g[(j, i)]

    detB = None
    for k in range(5):
        piv = G[(k, k)]
        detB = piv if detB is None else detB * piv
        inv = 1.0 / piv
        newG = {}
        for i in range(k + 1, 5):
            for j in range(i, 5):
                newG[(i, j)] = (gget(G, i, j)
                                - gget(G, i, k) * gget(G, k, j) * inv)
        G = newG

    A = {}
    for i in range(4):
        for j in range(i, 4):
            A[(i, j)] = (d[(i + 1, j + 1)] - d[(i + 1, 0)]
                         - d[(0, j + 1)] + d[(0, 0)])[None, :]
    detA = None
    for k in range(4):
        piv = A[(k, k)]
        detA = piv if detA is None else detA * piv
        inv = 1.0 / piv
        newA = {}
        for i in range(k + 1, 4):
            for j in range(i, 4):
                newA[(i, j)] = (gget(A, i, j)
                                - gget(A, i, k) * gget(A, k, j) * inv)
        A = newA

    out_ref[0] = -(detB / detA)


def _conv_branch(x, n_img, wmats, bvecs, gvecs, bevecs):
    xp = jnp.pad(x, ((0, 0), (0, 5), (1, 1), (1, 1)))
    xf = xp.reshape(n_img, 8, L1FLAT)
    xf = jnp.pad(xf, ((0, 0), (0, 0), (L1OFF, L1TOT - L1OFF - L1FLAT)))

    act1, st1 = pl.pallas_call(
        _l1_kernel,
        grid=(n_img,),
        in_specs=[
            pl.BlockSpec((1, 8, L1TOT), lambda i: (i, 0, 0)),
            pl.BlockSpec((72, 64), lambda i: (0, 0)),
            pl.BlockSpec((1, 64), lambda i: (0, 0)),
        ],
        out_specs=[
            pl.BlockSpec((1, IMG, IMG, 64), lambda i: (i, 0, 0, 0)),
            pl.BlockSpec((1, 2, 64), lambda i: (i, 0, 0)),
        ],
        out_shape=[
            jax.ShapeDtypeStruct((n_img, IMG, IMG, 64), F32),
            jax.ShapeDtypeStruct((n_img, 2, 64), F32),
        ],
        scratch_shapes=[pltpu.VMEM((72, L1LANES), F32),
                        pltpu.VMEM((L1G, L1G, 64), F32)],
        compiler_params=pltpu.CompilerParams(
            dimension_semantics=("parallel",),
            vmem_limit_bytes=56 * 1024 * 1024),
    )(xf, wmats[0], bvecs[0])

    geo = [(84, 42), (42, 21), (21, 10)]
    act, st = act1, st1
    for li, (h_in, h_out) in enumerate(geo):
        lk = functools.partial(_layer_kernel, n_img=n_img,
                               h_in=h_in, h_out=h_out)
        act, st = pl.pallas_call(
            lk,
            grid=(n_img,),
            in_specs=[
                pl.BlockSpec((1, h_in, h_in, 64), lambda i: (i, 0, 0, 0)),
                pl.BlockSpec((n_img, 2, 64), lambda i: (0, 0, 0)),
                pl.BlockSpec((576, 64), lambda i: (0, 0)),
                pl.BlockSpec((1, 64), lambda i: (0, 0)),
                pl.BlockSpec((1, 64), lambda i: (0, 0)),
                pl.BlockSpec((1, 64), lambda i: (0, 0)),
            ],
            out_specs=[
                pl.BlockSpec((1, h_out, h_out, 64), lambda i: (i, 0, 0, 0)),
                pl.BlockSpec((1, 2, 64), lambda i: (i, 0, 0)),
            ],
            out_shape=[
                jax.ShapeDtypeStruct((n_img, h_out, h_out, 64), F32),
                jax.ShapeDtypeStruct((n_img, 2, 64), F32),
            ],
            scratch_shapes=[
                pltpu.VMEM((h_out + 2, h_out + 2, 64), F32),
                pltpu.VMEM((h_out * h_out, 576), F32),
            ],
            compiler_params=pltpu.CompilerParams(
                dimension_semantics=("parallel",),
                vmem_limit_bytes=56 * 1024 * 1024),
        )(act, st, wmats[li + 1], bvecs[li + 1], gvecs[li], bevecs[li])

    hk = functools.partial(_head_kernel, n_img=n_img)
    feats = pl.pallas_call(
        hk,
        grid=(n_img,),
        in_specs=[
            pl.BlockSpec((1, 10, 10, 64), lambda i: (i, 0, 0, 0)),
            pl.BlockSpec((n_img, 2, 64), lambda i: (0, 0, 0)),
            pl.BlockSpec((1, 64), lambda i: (0, 0)),
            pl.BlockSpec((1, 64), lambda i: (0, 0)),
        ],
        out_specs=pl.BlockSpec((1, 25, 64), lambda i: (i, 0, 0)),
        out_shape=jax.ShapeDtypeStruct((n_img, 25, 64), F32),
        compiler_params=pltpu.CompilerParams(
            dimension_semantics=("parallel",),
            vmem_limit_bytes=56 * 1024 * 1024),
    )(act, st, gvecs[3], bevecs[3])
    return feats


def kernel(support, support_label, sample,
           w1, b1, g1, be1, w2, b2, g2, be2,
           w3, b3, g3, be3, w4, b4, g4, be4):
    del support_label
    w1t = jnp.transpose(w1, (2, 3, 1, 0))
    w1m = jnp.pad(w1t, ((0, 0), (0, 0), (0, 5), (0, 0))).reshape(72, 64)
    wmats = [w1m]
    for w in (w2, w3, w4):
        wmats.append(jnp.transpose(w, (2, 3, 1, 0)).reshape(576, 64))
    bvecs = [b.reshape(1, 64) for b in (b1, b2, b3, b4)]
    gvecs = [g.reshape(1, 64) for g in (g1, g2, g3, g4)]
    bevecs = [be.reshape(1, 64) for be in (be1, be2, be3, be4)]

    sup_x = support.reshape(N_SUP, 3, IMG, IMG)
    smp_x = sample.reshape(N_SMP, 3, IMG, IMG)
    sup_f = _conv_branch(sup_x, N_SUP, wmats, bvecs, gvecs, bevecs)
    smp_f = _conv_branch(smp_x, N_SMP, wmats, bvecs, gvecs, bevecs)

    sup_f = (sup_f.reshape(B, WAY, SHOT, 1600)
             .transpose(0, 2, 1, 3).reshape(B, WAY * SHOT, 1600))
    smp_f = smp_f.reshape(B, QUIRY, 1600)

    out = pl.pallas_call(
        _simplex_kernel,
        grid=(B,),
        in_specs=[
            pl.BlockSpec((1, WAY * SHOT, 1600), lambda b: (b, 0, 0)),
            pl.BlockSpec((1, QUIRY, 1600), lambda b: (b, 0, 0)),
        ],
        out_specs=pl.BlockSpec((1, QUIRY, WAY), lambda b: (b, 0, 0)),
        out_shape=jax.ShapeDtypeStruct((B, QUIRY, WAY), F32),
        compiler_params=pltpu.CompilerParams(
            dimension_semantics=("parallel",),
            vmem_limit_bytes=56 * 1024 * 1024),
    )(sup_f, smp_f)
    return out
